# baseline (device time: 66554 ns/iter reference)
import numpy as np

import jax
import jax.numpy as jnp
from jax import lax
from jax.experimental import pallas as pl
from jax.experimental.pallas import tpu as pltpu

N_DEV = 32
B, Sq, Skv, Hq, Dh = 2, 256, 256, 128, 64
H_LOC = Hq // N_DEV
D_MODEL = 512
ROWS = Sq // N_DEV
N_STAGES = 5

AR_ROWS = B * Sq
CH = AR_ROWS // N_DEV

_RS_OFF = {4: 0, 3: 16 * CH, 2: 24 * CH, 1: 28 * CH, 0: 30 * CH}

_qb = (np.arange(Sq) // 64)[:, None]
_kb = (np.arange(Skv) // 64)[None, :]
_MASK = (_qb == _kb) | (_kb == 0) | ((_qb + _kb) % 3 == 0)


def _ring_to_xyz(r):
    z = r // 8
    p = r % 8
    y = p // 2
    q = p % 2
    x = jnp.where(y % 2 == 0, q, 1 - q)
    return x, y, z


def _xyz_to_ring(x, y, z):
    return z * 8 + y * 2 + jnp.where(y % 2 == 0, x, 1 - x)


def _v_to_ring(v):
    x = (v // 16) % 2
    ylo = (v // 8) % 2
    zlo = (v // 4) % 2
    yhi = (v // 2) % 2
    zhi = v % 2
    return _xyz_to_ring(x, 2 * yhi + ylo, 2 * zhi + zlo)


def _flip_bit(v, k):
    bit = (v // (1 << k)) % 2
    return v + (1 - 2 * bit) * (1 << k)


def _allreduce_body(p_ref, out_ref, recv_rs, ss_rs, rs_rs, ss_ag, rs_ag):
    me = lax.axis_index("i")
    x, y, z = _ring_to_xyz(me)
    v = x * 16 + (y % 2) * 8 + (z % 2) * 4 + (y // 2) * 2 + (z // 2)

    partners = [_v_to_ring(_flip_bit(v, k)) for k in range(N_STAGES)]

    barrier = pltpu.get_barrier_semaphore()
    for pr in partners:
        pl.semaphore_signal(
            barrier, inc=1, device_id=(pr,),
            device_id_type=pl.DeviceIdType.MESH,
        )
    pl.semaphore_wait(barrier, N_STAGES)

    out_ref[...] = p_ref[...]

    for i, k in enumerate(reversed(range(N_STAGES))):
        n = 1 << k
        base = (v // (2 * n)) * (2 * n)
        bitk = (v // n) % 2
        keep = base + bitk * n
        send = base + (1 - bitk) * n
        rdma = pltpu.make_async_remote_copy(
            src_ref=out_ref.at[pl.ds(send * CH, n * CH), :],
            dst_ref=recv_rs.at[pl.ds(_RS_OFF[k], n * CH), :],
            send_sem=ss_rs.at[i],
            recv_sem=rs_rs.at[i],
            device_id=(partners[k],),
            device_id_type=pl.DeviceIdType.MESH,
        )
        rdma.start()
        rdma.wait()
        sl = pl.ds(keep * CH, n * CH)
        out_ref[sl, :] = out_ref[sl, :] + recv_rs[pl.ds(_RS_OFF[k], n * CH), :]

    for k in range(N_STAGES):
        n = 1 << k
        own = (v // n) * n
        sl = pl.ds(own * CH, n * CH)
        rdma = pltpu.make_async_remote_copy(
            src_ref=out_ref.at[sl, :],
            dst_ref=out_ref.at[sl, :],
            send_sem=ss_ag.at[k],
            recv_sem=rs_ag.at[k],
            device_id=(partners[k],),
            device_id_type=pl.DeviceIdType.MESH,
        )
        rdma.start()
        rdma.wait()


def _allreduce(p):
    return pl.pallas_call(
        _allreduce_body,
        out_shape=jax.ShapeDtypeStruct(p.shape, p.dtype),
        in_specs=[pl.BlockSpec(memory_space=pltpu.VMEM)],
        out_specs=pl.BlockSpec(memory_space=pltpu.VMEM),
        scratch_shapes=[
            pltpu.VMEM(((N_DEV - 1) * CH, D_MODEL), p.dtype),
            pltpu.SemaphoreType.DMA((N_STAGES,)),
            pltpu.SemaphoreType.DMA((N_STAGES,)),
            pltpu.SemaphoreType.DMA((N_STAGES,)),
            pltpu.SemaphoreType.DMA((N_STAGES,)),
        ],
        compiler_params=pltpu.CompilerParams(collective_id=0),
    )(p)


def kernel(x, Wq, K_ext, V_ext, Wo):
    me = lax.axis_index("i")

    bf16 = jnp.bfloat16
    Q = (x.astype(bf16) @ Wq.astype(bf16)).reshape(B, Sq, H_LOC, Dh)
    K = lax.dynamic_slice_in_dim(K_ext, me * H_LOC, H_LOC, axis=2).astype(bf16)
    V = lax.dynamic_slice_in_dim(V_ext, me * H_LOC, H_LOC, axis=2).astype(bf16)
    scores = jnp.einsum("bihd,bjhd->bhij", Q, K,
                        preferred_element_type=jnp.float32) * 0.125
    scores = jnp.where(jnp.asarray(_MASK)[None, None], scores, -1e9)
    w = jax.nn.softmax(scores, axis=-1).astype(bf16)
    ctx = jnp.einsum("bhij,bjhd->bihd", w, V,
                     preferred_element_type=bf16).reshape(B, Sq, H_LOC * Dh)
    partial = ctx @ Wo.astype(bf16)

    out16 = _allreduce(partial.reshape(AR_ROWS, D_MODEL))
    return out16.astype(jnp.float32).reshape(B, Sq, D_MODEL)


# device time: 64110 ns/iter; 1.0381x vs baseline; 1.0381x over previous
import numpy as np

import jax
import jax.numpy as jnp
from jax import lax
from jax.experimental import pallas as pl
from jax.experimental.pallas import tpu as pltpu

N_DEV = 32
B, Sq, Skv, Hq, Dh = 2, 256, 256, 128, 64
H_LOC = Hq // N_DEV
D_MODEL = 512
ROWS = Sq // N_DEV
N_STAGES = 5

AR_ROWS = B * Sq
CH = AR_ROWS // N_DEV

_RS_OFF = {4: 0, 3: 16 * CH, 2: 24 * CH}
_FUSED_OFF = {1: 28 * CH, 0: 32 * CH}
_RECV_CHUNKS = 36

_qb = (np.arange(Sq) // 64)[:, None]
_kb = (np.arange(Skv) // 64)[None, :]
_MASK = (_qb == _kb) | (_kb == 0) | ((_qb + _kb) % 3 == 0)


def _ring_to_xyz(r):
    z = r // 8
    p = r % 8
    y = p // 2
    q = p % 2
    x = jnp.where(y % 2 == 0, q, 1 - q)
    return x, y, z


def _xyz_to_ring(x, y, z):
    return z * 8 + y * 2 + jnp.where(y % 2 == 0, x, 1 - x)


def _v_to_ring(v):
    x = (v // 16) % 2
    ylo = (v // 8) % 2
    zlo = (v // 4) % 2
    yhi = (v // 2) % 2
    zhi = v % 2
    return _xyz_to_ring(x, 2 * yhi + ylo, 2 * zhi + zlo)


def _flip_bit(v, k):
    bit = (v // (1 << k)) % 2
    return v + (1 - 2 * bit) * (1 << k)


def _allreduce_body(p_ref, out_ref, recv_rs, ss_rs, rs_rs, ss_ag, rs_ag):
    me = lax.axis_index("i")
    x, y, z = _ring_to_xyz(me)
    v = x * 16 + (y % 2) * 8 + (z % 2) * 4 + (y // 2) * 2 + (z // 2)

    partners = [_v_to_ring(_flip_bit(v, k)) for k in range(N_STAGES)]

    barrier = pltpu.get_barrier_semaphore()
    for pr in partners:
        pl.semaphore_signal(
            barrier, inc=1, device_id=(pr,),
            device_id_type=pl.DeviceIdType.MESH,
        )
    pl.semaphore_wait(barrier, N_STAGES)

    out_ref[...] = p_ref[...]

    for i, k in enumerate([4, 3, 2]):
        n = 1 << k
        base = (v // (2 * n)) * (2 * n)
        bitk = (v // n) % 2
        keep = base + bitk * n
        send = base + (1 - bitk) * n
        rdma = pltpu.make_async_remote_copy(
            src_ref=out_ref.at[pl.ds(send * CH, n * CH), :],
            dst_ref=recv_rs.at[pl.ds(_RS_OFF[k], n * CH), :],
            send_sem=ss_rs.at[i],
            recv_sem=rs_rs.at[i],
            device_id=(partners[k],),
            device_id_type=pl.DeviceIdType.MESH,
        )
        rdma.start()
        rdma.wait()
        sl = pl.ds(keep * CH, n * CH)
        out_ref[sl, :] = out_ref[sl, :] + recv_rs[pl.ds(_RS_OFF[k], n * CH), :]

    qbase = (v // 4) * 4
    sl_q = pl.ds(qbase * CH, 4 * CH)
    for j, k in enumerate([1, 0]):
        rdma = pltpu.make_async_remote_copy(
            src_ref=out_ref.at[sl_q, :],
            dst_ref=recv_rs.at[pl.ds(_FUSED_OFF[k], 4 * CH), :],
            send_sem=ss_rs.at[3 + j],
            recv_sem=rs_rs.at[3 + j],
            device_id=(partners[k],),
            device_id_type=pl.DeviceIdType.MESH,
        )
        rdma.start()
        rdma.wait()
        out_ref[sl_q, :] = out_ref[sl_q, :] + recv_rs[pl.ds(_FUSED_OFF[k], 4 * CH), :]

    for k in [2, 3, 4]:
        n = 1 << k
        own = (v // n) * n
        sl = pl.ds(own * CH, n * CH)
        rdma = pltpu.make_async_remote_copy(
            src_ref=out_ref.at[sl, :],
            dst_ref=out_ref.at[sl, :],
            send_sem=ss_ag.at[k],
            recv_sem=rs_ag.at[k],
            device_id=(partners[k],),
            device_id_type=pl.DeviceIdType.MESH,
        )
        rdma.start()
        rdma.wait()


def _allreduce(p):
    return pl.pallas_call(
        _allreduce_body,
        out_shape=jax.ShapeDtypeStruct(p.shape, p.dtype),
        in_specs=[pl.BlockSpec(memory_space=pltpu.VMEM)],
        out_specs=pl.BlockSpec(memory_space=pltpu.VMEM),
        scratch_shapes=[
            pltpu.VMEM((_RECV_CHUNKS * CH, D_MODEL), p.dtype),
            pltpu.SemaphoreType.DMA((N_STAGES,)),
            pltpu.SemaphoreType.DMA((N_STAGES,)),
            pltpu.SemaphoreType.DMA((N_STAGES,)),
            pltpu.SemaphoreType.DMA((N_STAGES,)),
        ],
        compiler_params=pltpu.CompilerParams(collective_id=0),
    )(p)


def kernel(x, Wq, K_ext, V_ext, Wo):
    me = lax.axis_index("i")

    Q = (x @ Wq).reshape(B, Sq, H_LOC, Dh)
    K = lax.dynamic_slice_in_dim(K_ext, me * H_LOC, H_LOC, axis=2)
    V = lax.dynamic_slice_in_dim(V_ext, me * H_LOC, H_LOC, axis=2)
    scores = jnp.einsum("bihd,bjhd->bhij", Q, K) * 0.125
    scores = jnp.where(jnp.asarray(_MASK)[None, None], scores, -1e9)
    w = jax.nn.softmax(scores, axis=-1)
    ctx = jnp.einsum("bhij,bjhd->bihd", w, V).reshape(B, Sq, H_LOC * Dh)
    partial = (ctx @ Wo).astype(jnp.bfloat16)

    out16 = _allreduce(partial.reshape(AR_ROWS, D_MODEL))
    return out16.astype(jnp.float32).reshape(B, Sq, D_MODEL)


# device time: 63998 ns/iter; 1.0399x vs baseline; 1.0018x over previous
import numpy as np

import jax
import jax.numpy as jnp
from jax import lax
from jax.experimental import pallas as pl
from jax.experimental.pallas import tpu as pltpu

N_DEV = 32
B, Sq, Skv, Hq, Dh = 2, 256, 256, 128, 64
H_LOC = Hq // N_DEV
D_MODEL = 512
ROWS = Sq // N_DEV
N_STAGES = 5

AR_ROWS = B * Sq
CH = AR_ROWS // N_DEV

_RS_OFF = {4: 0, 3: 16 * CH, 2: 24 * CH}
_FUSED_OFF = {1: 28 * CH, 0: 32 * CH}
_RECV_CHUNKS = 36

_qb = (np.arange(Sq) // 64)[:, None]
_kb = (np.arange(Skv) // 64)[None, :]
_MASK = (_qb == _kb) | (_kb == 0) | ((_qb + _kb) % 3 == 0)


def _ring_to_xyz(r):
    z = r // 8
    p = r % 8
    y = p // 2
    q = p % 2
    x = jnp.where(y % 2 == 0, q, 1 - q)
    return x, y, z


def _xyz_to_ring(x, y, z):
    return z * 8 + y * 2 + jnp.where(y % 2 == 0, x, 1 - x)


def _v_to_ring(v):
    x = (v // 16) % 2
    ylo = (v // 8) % 2
    zlo = (v // 4) % 2
    yhi = (v // 2) % 2
    zhi = v % 2
    return _xyz_to_ring(x, 2 * yhi + ylo, 2 * zhi + zlo)


def _flip_bit(v, k):
    bit = (v // (1 << k)) % 2
    return v + (1 - 2 * bit) * (1 << k)


def _allreduce_body(p_ref, out_ref, recv_rs, ss_rs, rs_rs, ss_ag, rs_ag):
    me = lax.axis_index("i")
    x, y, z = _ring_to_xyz(me)
    v = x * 16 + (y % 2) * 8 + (z % 2) * 4 + (y // 2) * 2 + (z // 2)

    partners = [_v_to_ring(_flip_bit(v, k)) for k in range(N_STAGES)]

    barrier = pltpu.get_barrier_semaphore()
    for pr in partners:
        pl.semaphore_signal(
            barrier, inc=1, device_id=(pr,),
            device_id_type=pl.DeviceIdType.MESH,
        )
    pl.semaphore_wait(barrier, N_STAGES)

    out_ref[...] = p_ref[...]

    for i, k in enumerate([4, 3, 2]):
        n = 1 << k
        base = (v // (2 * n)) * (2 * n)
        bitk = (v // n) % 2
        keep = base + bitk * n
        send = base + (1 - bitk) * n
        rdma = pltpu.make_async_remote_copy(
            src_ref=out_ref.at[pl.ds(send * CH, n * CH), :],
            dst_ref=recv_rs.at[pl.ds(_RS_OFF[k], n * CH), :],
            send_sem=ss_rs.at[i],
            recv_sem=rs_rs.at[i],
            device_id=(partners[k],),
            device_id_type=pl.DeviceIdType.MESH,
        )
        rdma.start()
        rdma.wait()
        sl = pl.ds(keep * CH, n * CH)
        out_ref[sl, :] = out_ref[sl, :] + recv_rs[pl.ds(_RS_OFF[k], n * CH), :]

    qbase = (v // 4) * 4
    sl_q = pl.ds(qbase * CH, 4 * CH)
    for j, k in enumerate([1, 0]):
        rdma = pltpu.make_async_remote_copy(
            src_ref=out_ref.at[sl_q, :],
            dst_ref=recv_rs.at[pl.ds(_FUSED_OFF[k], 4 * CH), :],
            send_sem=ss_rs.at[3 + j],
            recv_sem=rs_rs.at[3 + j],
            device_id=(partners[k],),
            device_id_type=pl.DeviceIdType.MESH,
        )
        rdma.start()
        rdma.wait()
        out_ref[sl_q, :] = out_ref[sl_q, :] + recv_rs[pl.ds(_FUSED_OFF[k], 4 * CH), :]

    for k in [2, 3, 4]:
        n = 1 << k
        own = (v // n) * n
        sl = pl.ds(own * CH, n * CH)
        rdma = pltpu.make_async_remote_copy(
            src_ref=out_ref.at[sl, :],
            dst_ref=out_ref.at[sl, :],
            send_sem=ss_ag.at[k],
            recv_sem=rs_ag.at[k],
            device_id=(partners[k],),
            device_id_type=pl.DeviceIdType.MESH,
        )
        rdma.start()
        rdma.wait()


def _allreduce(p):
    return pl.pallas_call(
        _allreduce_body,
        out_shape=jax.ShapeDtypeStruct(p.shape, p.dtype),
        in_specs=[pl.BlockSpec(memory_space=pltpu.VMEM)],
        out_specs=pl.BlockSpec(memory_space=pltpu.VMEM),
        scratch_shapes=[
            pltpu.VMEM((_RECV_CHUNKS * CH, D_MODEL), p.dtype),
            pltpu.SemaphoreType.DMA((N_STAGES,)),
            pltpu.SemaphoreType.DMA((N_STAGES,)),
            pltpu.SemaphoreType.DMA((N_STAGES,)),
            pltpu.SemaphoreType.DMA((N_STAGES,)),
        ],
        compiler_params=pltpu.CompilerParams(collective_id=0),
    )(p)


def kernel(x, Wq, K_ext, V_ext, Wo):
    me = lax.axis_index("i")

    bf16 = jnp.bfloat16
    Q2 = jnp.transpose((x @ Wq).reshape(B, Sq, H_LOC, Dh),
                       (0, 2, 1, 3)).astype(bf16)
    K2 = jnp.transpose(
        lax.dynamic_slice_in_dim(K_ext, me * H_LOC, H_LOC, axis=2),
        (0, 2, 3, 1)).astype(bf16)
    V2 = jnp.transpose(
        lax.dynamic_slice_in_dim(V_ext, me * H_LOC, H_LOC, axis=2),
        (0, 2, 1, 3)).astype(bf16)
    scores = jnp.einsum("bhid,bhdj->bhij", Q2, K2,
                        preferred_element_type=jnp.float32) * 0.125
    scores = jnp.where(jnp.asarray(_MASK)[None, None], scores, -1e9)
    w = jax.nn.softmax(scores, axis=-1).astype(bf16)
    ctx = jnp.einsum("bhij,bhjd->bhid", w, V2,
                     preferred_element_type=bf16)
    ctx = jnp.transpose(ctx, (0, 2, 1, 3)).reshape(B, Sq, H_LOC * Dh)
    partial = ctx @ Wo.astype(bf16)

    out16 = _allreduce(partial.reshape(AR_ROWS, D_MODEL))
    return out16.astype(jnp.float32).reshape(B, Sq, D_MODEL)


# device time: 59602 ns/iter; 1.1166x vs baseline; 1.0738x over previous
import numpy as np

import jax
import jax.numpy as jnp
from jax import lax
from jax.experimental import pallas as pl
from jax.experimental.pallas import tpu as pltpu

N_DEV = 32
B, Sq, Skv, Hq, Dh = 2, 256, 256, 128, 64
H_LOC = Hq // N_DEV
D_MODEL = 512
ROWS = Sq // N_DEV
N_STAGES = 5

AR_ROWS = B * Sq
CH = AR_ROWS // N_DEV

HALF = D_MODEL // 2
_RECV_OFF = [0, 16 * CH, 24 * CH, 28 * CH, 32 * CH]
_RECV_CHUNKS = 36

_qb = (np.arange(Sq) // 64)[:, None]
_kb = (np.arange(Skv) // 64)[None, :]
_MASK = (_qb == _kb) | (_kb == 0) | ((_qb + _kb) % 3 == 0)


def _ring_to_xyz(r):
    z = r // 8
    p = r % 8
    y = p // 2
    q = p % 2
    x = jnp.where(y % 2 == 0, q, 1 - q)
    return x, y, z


def _xyz_to_ring(x, y, z):
    return z * 8 + y * 2 + jnp.where(y % 2 == 0, x, 1 - x)


def _v_to_ring(v):
    x = (v // 16) % 2
    ylo = (v // 8) % 2
    zlo = (v // 4) % 2
    yhi = (v // 2) % 2
    zhi = v % 2
    return _xyz_to_ring(x, 2 * yhi + ylo, 2 * zhi + zlo)


def _flip_bit(v, k):
    bit = (v // (1 << k)) % 2
    return v + (1 - 2 * bit) * (1 << k)


def _stage_geometry(vb, stage):
    if stage <= 2:
        n = 1 << (4 - stage)
        base = (vb // (2 * n)) * (2 * n)
        bitk = (vb // n) % 2
        return (base + (1 - bitk) * n) * CH, n, (base + bitk * n) * CH
    if stage <= 4:
        qbase = (vb // 4) * 4
        return qbase * CH, 4, qbase * CH
    n = 1 << (stage - 3)
    return (vb // n) * n * CH, n, None


def _allreduce_body(p_ref, out_ref, recv_rs, ssA, rsA, ssB, rsB):
    me = lax.axis_index("i")
    x, y, z = _ring_to_xyz(me)
    ylo, yhi, zlo, zhi = y % 2, y // 2, z % 2, z // 2
    vA = x * 16 + ylo * 8 + zlo * 4 + yhi * 2 + zhi
    vB = ylo * 16 + zlo * 8 + x * 4 + zhi * 2 + yhi

    P = [_v_to_ring(_flip_bit(vA, k)) for k in range(N_STAGES)]
    pA = [P[4], P[3], P[2], P[1], P[0], P[2], P[3], P[4]]
    pB = [P[3], P[2], P[4], P[0], P[1], P[4], P[2], P[3]]

    barrier = pltpu.get_barrier_semaphore()
    for pr in P:
        pl.semaphore_signal(
            barrier, inc=1, device_id=(pr,),
            device_id_type=pl.DeviceIdType.MESH,
        )
    pl.semaphore_wait(barrier, N_STAGES)

    out_ref[...] = p_ref[...]

    colA = pl.ds(0, HALF)
    colB = pl.ds(HALF, HALF)
    for s in range(8):
        srcA, nA, addA = _stage_geometry(vA, s)
        srcB, nB, addB = _stage_geometry(vB, s)
        roff = _RECV_OFF[s] if s <= 4 else None
        dA = pltpu.make_async_remote_copy(
            src_ref=out_ref.at[pl.ds(srcA, nA * CH), colA],
            dst_ref=(recv_rs.at[pl.ds(roff, nA * CH), colA]
                     if roff is not None
                     else out_ref.at[pl.ds(srcA, nA * CH), colA]),
            send_sem=ssA.at[s], recv_sem=rsA.at[s],
            device_id=(pA[s],), device_id_type=pl.DeviceIdType.MESH,
        )
        dB = pltpu.make_async_remote_copy(
            src_ref=out_ref.at[pl.ds(srcB, nB * CH), colB],
            dst_ref=(recv_rs.at[pl.ds(roff, nB * CH), colB]
                     if roff is not None
                     else out_ref.at[pl.ds(srcB, nB * CH), colB]),
            send_sem=ssB.at[s], recv_sem=rsB.at[s],
            device_id=(pB[s],), device_id_type=pl.DeviceIdType.MESH,
        )
        dA.start()
        dB.start()
        dA.wait()
        if addA is not None:
            slA = pl.ds(addA, nA * CH)
            out_ref[slA, colA] = (
                out_ref[slA, colA] + recv_rs[pl.ds(roff, nA * CH), colA])
        dB.wait()
        if addB is not None:
            slB = pl.ds(addB, nB * CH)
            out_ref[slB, colB] = (
                out_ref[slB, colB] + recv_rs[pl.ds(roff, nB * CH), colB])


def _allreduce(p):
    return pl.pallas_call(
        _allreduce_body,
        out_shape=jax.ShapeDtypeStruct(p.shape, p.dtype),
        in_specs=[pl.BlockSpec(memory_space=pltpu.VMEM)],
        out_specs=pl.BlockSpec(memory_space=pltpu.VMEM),
        scratch_shapes=[
            pltpu.VMEM((_RECV_CHUNKS * CH, D_MODEL), p.dtype),
            pltpu.SemaphoreType.DMA((8,)),
            pltpu.SemaphoreType.DMA((8,)),
            pltpu.SemaphoreType.DMA((8,)),
            pltpu.SemaphoreType.DMA((8,)),
        ],
        compiler_params=pltpu.CompilerParams(collective_id=0),
    )(p)


def kernel(x, Wq, K_ext, V_ext, Wo):
    me = lax.axis_index("i")

    bf16 = jnp.bfloat16
    Q2 = jnp.transpose((x @ Wq).reshape(B, Sq, H_LOC, Dh),
                       (0, 2, 1, 3)).astype(bf16)
    K2 = jnp.transpose(
        lax.dynamic_slice_in_dim(K_ext, me * H_LOC, H_LOC, axis=2),
        (0, 2, 3, 1)).astype(bf16)
    V2 = jnp.transpose(
        lax.dynamic_slice_in_dim(V_ext, me * H_LOC, H_LOC, axis=2),
        (0, 2, 1, 3)).astype(bf16)
    scores = jnp.einsum("bhid,bhdj->bhij", Q2, K2,
                        preferred_element_type=jnp.float32) * 0.125
    scores = jnp.where(jnp.asarray(_MASK)[None, None], scores, -1e9)
    w = jax.nn.softmax(scores, axis=-1).astype(bf16)
    ctx = jnp.einsum("bhij,bhjd->bhid", w, V2,
                     preferred_element_type=bf16)
    ctx = jnp.transpose(ctx, (0, 2, 1, 3)).reshape(B, Sq, H_LOC * Dh)
    partial = ctx @ Wo.astype(bf16)

    out16 = _allreduce(partial.reshape(AR_ROWS, D_MODEL))
    return out16.astype(jnp.float32).reshape(B, Sq, D_MODEL)


# device time: 58328 ns/iter; 1.1410x vs baseline; 1.0218x over previous
import numpy as np

import jax
import jax.numpy as jnp
from jax import lax
from jax.experimental import pallas as pl
from jax.experimental.pallas import tpu as pltpu

N_DEV = 32
B, Sq, Skv, Hq, Dh = 2, 256, 256, 128, 64
H_LOC = Hq // N_DEV
D_MODEL = 512
N_STAGES = 5

AR_ROWS = B * Sq
CH = AR_ROWS // N_DEV

HALF = D_MODEL // 2
_RECV_OFF = [0, 16 * CH, 24 * CH, 28 * CH, 32 * CH]
_RECV_CHUNKS = 36

_qb = (np.arange(Sq) // 64)[:, None]
_kb = (np.arange(Skv) // 64)[None, :]
_MASK = (_qb == _kb) | (_kb == 0) | ((_qb + _kb) % 3 == 0)


def _ring_to_xyz(r):
    z = r // 8
    p = r % 8
    y = p // 2
    q = p % 2
    x = jnp.where(y % 2 == 0, q, 1 - q)
    return x, y, z


def _xyz_to_ring(x, y, z):
    return z * 8 + y * 2 + jnp.where(y % 2 == 0, x, 1 - x)


def _v_to_ring(v):
    x = (v // 16) % 2
    ylo = (v // 8) % 2
    zlo = (v // 4) % 2
    yhi = (v // 2) % 2
    zhi = v % 2
    return _xyz_to_ring(x, 2 * yhi + ylo, 2 * zhi + zlo)


def _flip_bit(v, k):
    bit = (v // (1 << k)) % 2
    return v + (1 - 2 * bit) * (1 << k)


def _stage_geometry(vb, stage):
    if stage <= 2:
        n = 1 << (4 - stage)
        base = (vb // (2 * n)) * (2 * n)
        bitk = (vb // n) % 2
        return (base + (1 - bitk) * n) * CH, n, (base + bitk * n) * CH
    if stage <= 4:
        qbase = (vb // 4) * 4
        return qbase * CH, 4, qbase * CH
    n = 1 << (stage - 3)
    return (vb // n) * n * CH, n, None


def _allreduce_body(p_ref, out_ref, recv_rs, ssA, rsA, ssB, rsB):
    me = lax.axis_index("i")
    x, y, z = _ring_to_xyz(me)
    ylo, yhi, zlo, zhi = y % 2, y // 2, z % 2, z // 2
    vA = x * 16 + ylo * 8 + zlo * 4 + yhi * 2 + zhi
    vB = ylo * 16 + zlo * 8 + x * 4 + zhi * 2 + yhi

    P = [_v_to_ring(_flip_bit(vA, k)) for k in range(N_STAGES)]
    pA = [P[4], P[3], P[2], P[1], P[0], P[2], P[3], P[4]]
    pB = [P[3], P[2], P[4], P[0], P[1], P[4], P[2], P[3]]

    barrier = pltpu.get_barrier_semaphore()
    for pr in P:
        pl.semaphore_signal(
            barrier, inc=1, device_id=(pr,),
            device_id_type=pl.DeviceIdType.MESH,
        )
    pl.semaphore_wait(barrier, N_STAGES)

    out_ref[...] = p_ref[...]

    colA = pl.ds(0, HALF)
    colB = pl.ds(HALF, HALF)
    for s in range(8):
        srcA, nA, addA = _stage_geometry(vA, s)
        srcB, nB, addB = _stage_geometry(vB, s)
        roff = _RECV_OFF[s] if s <= 4 else None
        dA = pltpu.make_async_remote_copy(
            src_ref=out_ref.at[pl.ds(srcA, nA * CH), colA],
            dst_ref=(recv_rs.at[pl.ds(roff, nA * CH), colA]
                     if roff is not None
                     else out_ref.at[pl.ds(srcA, nA * CH), colA]),
            send_sem=ssA.at[s], recv_sem=rsA.at[s],
            device_id=(pA[s],), device_id_type=pl.DeviceIdType.MESH,
        )
        dB = pltpu.make_async_remote_copy(
            src_ref=out_ref.at[pl.ds(srcB, nB * CH), colB],
            dst_ref=(recv_rs.at[pl.ds(roff, nB * CH), colB]
                     if roff is not None
                     else out_ref.at[pl.ds(srcB, nB * CH), colB]),
            send_sem=ssB.at[s], recv_sem=rsB.at[s],
            device_id=(pB[s],), device_id_type=pl.DeviceIdType.MESH,
        )
        dA.start()
        dB.start()
        dA.wait()
        if addA is not None:
            slA = pl.ds(addA, nA * CH)
            out_ref[slA, colA] = (
                out_ref[slA, colA] + recv_rs[pl.ds(roff, nA * CH), colA])
        dB.wait()
        if addB is not None:
            slB = pl.ds(addB, nB * CH)
            out_ref[slB, colB] = (
                out_ref[slB, colB] + recv_rs[pl.ds(roff, nB * CH), colB])


def _allreduce(p):
    return pl.pallas_call(
        _allreduce_body,
        out_shape=jax.ShapeDtypeStruct(p.shape, p.dtype),
        in_specs=[pl.BlockSpec(memory_space=pltpu.VMEM)],
        out_specs=pl.BlockSpec(memory_space=pltpu.VMEM),
        scratch_shapes=[
            pltpu.VMEM((_RECV_CHUNKS * CH, D_MODEL), p.dtype),
            pltpu.SemaphoreType.DMA((8,)),
            pltpu.SemaphoreType.DMA((8,)),
            pltpu.SemaphoreType.DMA((8,)),
            pltpu.SemaphoreType.DMA((8,)),
        ],
        compiler_params=pltpu.CompilerParams(collective_id=0),
    )(p)


def kernel(x, Wq, K_ext, V_ext, Wo):
    me = lax.axis_index("i")

    bf16 = jnp.bfloat16
    Q2 = jnp.transpose((x @ Wq).reshape(B, Sq, H_LOC, Dh),
                       (0, 2, 1, 3)).astype(bf16)
    K2 = jnp.transpose(
        lax.dynamic_slice_in_dim(K_ext, me * H_LOC, H_LOC, axis=2),
        (0, 2, 3, 1)).astype(bf16)
    V2 = jnp.transpose(
        lax.dynamic_slice_in_dim(V_ext, me * H_LOC, H_LOC, axis=2),
        (0, 2, 1, 3)).astype(bf16)
    scores = jnp.einsum("bhid,bhdj->bhij", Q2, K2,
                        preferred_element_type=jnp.float32) * 0.125
    scores = jnp.where(jnp.asarray(_MASK)[None, None], scores, -1e9)
    w = jax.nn.softmax(scores, axis=-1).astype(bf16)
    ctx = jnp.einsum("bhij,bhjd->bhid", w, V2,
                     preferred_element_type=bf16)
    ctx = jnp.transpose(ctx, (0, 2, 1, 3)).reshape(B, Sq, H_LOC * Dh)
    partial = ctx @ Wo.astype(bf16)

    out16 = _allreduce(partial.reshape(AR_ROWS, D_MODEL))
    return out16.astype(jnp.float32).reshape(B, Sq, D_MODEL)
